# baseline (speedup 1.0000x reference)
import functools

import jax
import jax.numpy as jnp
from jax.experimental import pallas as pl
from jax.experimental.pallas import tpu as pltpu

_ALPHA = 0.2
_PAD = 8


def _proj_body(x_ref, w_ref, ab_ref, h_ref, hpack_ref, rowfac_ref,
               colfac_ref, hsum_ref, *, heads, hdim):
    i = pl.program_id(0)
    h = jnp.dot(x_ref[...], w_ref[...], preferred_element_type=jnp.float32)
    h_ref[...] = h
    tile = h.shape[0]
    g = jnp.dot(h, ab_ref[...], preferred_element_type=jnp.float32)
    m = g.shape[1] // 2
    gs, gd = g[:, :m], g[:, m:]
    rowfac_ref[...] = jnp.exp((_ALPHA - 1.0) * gs).astype(jnp.bfloat16)
    colfac_ref[...] = jnp.exp((1.0 - _ALPHA) * gd).astype(jnp.bfloat16).T
    fd = jnp.exp(_ALPHA * gd)
    pieces = []
    zeros = jnp.zeros((tile, _PAD - 1), dtype=jnp.float32)
    for k in range(heads):
        fdk = fd[:, k:k + 1]
        pieces += [h[:, k * hdim:(k + 1) * hdim] * fdk, fdk, zeros]
    hpack_ref[...] = jnp.concatenate(pieces, axis=1).astype(jnp.bfloat16)

    @pl.when(i == 0)
    def _init():
        hsum_ref[...] = jnp.zeros_like(hsum_ref)

    hsum_ref[...] += jnp.sum(h, axis=0, keepdims=True)


def _project(x, w, ab, heads, hdim, tile):
    n, f = x.shape
    d = w.shape[1]
    m2 = ab.shape[1]
    dp = heads * (hdim + _PAD)
    body = functools.partial(_proj_body, heads=heads, hdim=hdim)
    return pl.pallas_call(
        body,
        grid=(n // tile,),
        in_specs=[
            pl.BlockSpec((tile, f), lambda i: (i, 0)),
            pl.BlockSpec((f, d), lambda i: (0, 0)),
            pl.BlockSpec((d, m2), lambda i: (0, 0)),
        ],
        out_specs=[
            pl.BlockSpec((tile, d), lambda i: (i, 0)),
            pl.BlockSpec((tile, dp), lambda i: (i, 0)),
            pl.BlockSpec((tile, m2 // 2), lambda i: (i, 0)),
            pl.BlockSpec((m2 // 2, tile), lambda i: (0, i)),
            pl.BlockSpec((1, d), lambda i: (0, 0)),
        ],
        out_shape=[
            jax.ShapeDtypeStruct((n, d), jnp.float32),
            jax.ShapeDtypeStruct((n, dp), jnp.bfloat16),
            jax.ShapeDtypeStruct((n, m2 // 2), jnp.bfloat16),
            jax.ShapeDtypeStruct((m2 // 2, n), jnp.bfloat16),
            jax.ShapeDtypeStruct((1, d), jnp.float32),
        ],
    )(x, w, ab)


def _att_body(adj_ref, rowfac_ref, colfac_ref, hpack_ref, hsum_ref, o_ref,
              *maybe_mask_ref, heads, hdim, relu):
    adjb = adj_ref[...].astype(jnp.bfloat16)
    if maybe_mask_ref:
        maybe_mask_ref[0][...] = adjb.astype(jnp.float8_e5m2)
    n_src = adj_ref.shape[1]
    stride = hdim + _PAD
    for k in range(heads):
        r = rowfac_ref[:, k:k + 1]
        q = colfac_ref[k:k + 1, :]
        w = jnp.maximum(q, r) * adjb
        acc = jnp.dot(w, hpack_ref[:, k * stride:k * stride + hdim + 1],
                      preferred_element_type=jnp.float32)
        num, z = acc[:, :hdim], acc[:, hdim:hdim + 1]
        hmean = hsum_ref[:, k * hdim:(k + 1) * hdim] * (1.0 / n_src)
        out = jnp.where(z > 0, num / z, hmean)
        if relu:
            out = jnp.maximum(out, 0.0)
        o_ref[:, k * hdim:(k + 1) * hdim] = out


def _attention(adj, rowfac, colfac, hpack, hsum, heads, hdim, relu, tile,
               emit_mask=False):
    n = adj.shape[0]
    m2 = colfac.shape[0]
    mh = rowfac.shape[1]
    dp = hpack.shape[1]
    d = heads * hdim
    body = functools.partial(_att_body, heads=heads, hdim=hdim, relu=relu)
    out_specs = [pl.BlockSpec((tile, d), lambda i: (i, 0))]
    out_shape = [jax.ShapeDtypeStruct((n, d), jnp.float32)]
    if emit_mask:
        out_specs.append(pl.BlockSpec((tile, n), lambda i: (i, 0)))
        out_shape.append(jax.ShapeDtypeStruct((n, n), jnp.float8_e5m2))
    res = pl.pallas_call(
        body,
        grid=(n // tile,),
        in_specs=[
            pl.BlockSpec((tile, n), lambda i: (i, 0)),
            pl.BlockSpec((tile, mh), lambda i: (i, 0)),
            pl.BlockSpec((m2, n), lambda i: (0, 0)),
            pl.BlockSpec((n, dp), lambda i: (0, 0)),
            pl.BlockSpec((1, d), lambda i: (0, 0)),
        ],
        out_specs=out_specs,
        out_shape=out_shape,
        compiler_params=pltpu.CompilerParams(
            dimension_semantics=("parallel",)),
    )(adj, rowfac, colfac, hpack, hsum)
    return res if emit_mask else (res[0], None)


def kernel(x, adj, W1, a1_src, a1_dst, W2, a2_src, a2_dst):
    K, F_IN, H = W1.shape
    C = W2.shape[2]

    w1c = jnp.transpose(W1, (1, 0, 2)).reshape(F_IN, K * H)
    eye_k = jnp.eye(K, dtype=jnp.float32)
    A1 = jnp.einsum('ko,kj->koj', a1_src, eye_k).reshape(K * H, K)
    B1 = jnp.einsum('ko,kj->koj', a1_dst, eye_k).reshape(K * H, K)
    ab1 = jnp.concatenate([A1, B1], axis=1)
    w2c = W2.reshape(K * H, C)
    ab2 = jnp.concatenate([a2_src.T, a2_dst.T], axis=1)

    _, hp1, rf1, cf1, hs1 = _project(x, w1c, ab1, heads=K, hdim=H, tile=512)
    o1, mask8 = _attention(adj, rf1, cf1, hp1, hs1, heads=K, hdim=H,
                           relu=True, tile=1024, emit_mask=True)
    _, hp2, rf2, cf2, hs2 = _project(o1, w2c, ab2, heads=1, hdim=C, tile=512)
    out, _ = _attention(mask8, rf2, cf2, hp2, hs2, heads=1, hdim=C,
                        relu=False, tile=1024)
    return out


# speedup vs baseline: 1.0489x; 1.0489x over previous
import functools

import jax
import jax.numpy as jnp
from jax.experimental import pallas as pl
from jax.experimental.pallas import tpu as pltpu

_ALPHA = 0.2
_STRIDE = 128


def _proj_body(x_ref, w_ref, ab_ref, h_ref, hpack_ref, rowfac_ref,
               colfac_ref, hsum_ref, *, heads, hdim):
    i = pl.program_id(0)
    h = jnp.dot(x_ref[...], w_ref[...], preferred_element_type=jnp.float32)
    h_ref[...] = h
    tile = h.shape[0]
    g = jnp.dot(h, ab_ref[...], preferred_element_type=jnp.float32)
    m = g.shape[1] // 2
    gs, gd = g[:, :m], g[:, m:]
    rowfac_ref[...] = jnp.exp((_ALPHA - 1.0) * gs).astype(jnp.bfloat16)
    colfac_ref[...] = jnp.exp((1.0 - _ALPHA) * gd).astype(jnp.bfloat16).T
    fd = jnp.exp(_ALPHA * gd)
    pieces = []
    zeros = jnp.zeros((tile, _STRIDE - hdim - 1), dtype=jnp.float32)
    for k in range(heads):
        fdk = fd[:, k:k + 1]
        pieces += [h[:, k * hdim:(k + 1) * hdim] * fdk, fdk, zeros]
    hpack_ref[...] = jnp.concatenate(pieces, axis=1).astype(jnp.bfloat16)

    @pl.when(i == 0)
    def _init():
        hsum_ref[...] = jnp.zeros_like(hsum_ref)

    hsum_ref[...] += jnp.sum(h, axis=0, keepdims=True)


def _project(x, w, ab, heads, hdim, tile):
    n, f = x.shape
    d = w.shape[1]
    m2 = ab.shape[1]
    dp = heads * _STRIDE
    body = functools.partial(_proj_body, heads=heads, hdim=hdim)
    return pl.pallas_call(
        body,
        grid=(n // tile,),
        in_specs=[
            pl.BlockSpec((tile, f), lambda i: (i, 0)),
            pl.BlockSpec((f, d), lambda i: (0, 0)),
            pl.BlockSpec((d, m2), lambda i: (0, 0)),
        ],
        out_specs=[
            pl.BlockSpec((tile, d), lambda i: (i, 0)),
            pl.BlockSpec((tile, dp), lambda i: (i, 0)),
            pl.BlockSpec((tile, m2 // 2), lambda i: (i, 0)),
            pl.BlockSpec((m2 // 2, tile), lambda i: (0, i)),
            pl.BlockSpec((1, d), lambda i: (0, 0)),
        ],
        out_shape=[
            jax.ShapeDtypeStruct((n, d), jnp.float32),
            jax.ShapeDtypeStruct((n, dp), jnp.bfloat16),
            jax.ShapeDtypeStruct((n, m2 // 2), jnp.bfloat16),
            jax.ShapeDtypeStruct((m2 // 2, n), jnp.bfloat16),
            jax.ShapeDtypeStruct((1, d), jnp.float32),
        ],
    )(x, w, ab)


def _att_body(adj_ref, rowfac_ref, colfac_ref, hpack_ref, hsum_ref, o_ref,
              *maybe_mask_ref, heads, hdim, relu):
    adjb = adj_ref[...].astype(jnp.bfloat16)
    if maybe_mask_ref:
        maybe_mask_ref[0][...] = adjb.astype(jnp.float8_e5m2)
    n_src = adj_ref.shape[1]
    stride = _STRIDE
    for k in range(heads):
        r = rowfac_ref[:, k:k + 1]
        q = colfac_ref[k:k + 1, :]
        w = jnp.maximum(q, r) * adjb
        acc = jnp.dot(w, hpack_ref[:, k * stride:k * stride + hdim + 1],
                      preferred_element_type=jnp.float32)
        num, z = acc[:, :hdim], acc[:, hdim:hdim + 1]
        hmean = hsum_ref[:, k * hdim:(k + 1) * hdim] * (1.0 / n_src)
        out = jnp.where(z > 0, num / z, hmean)
        if relu:
            out = jnp.maximum(out, 0.0)
        o_ref[:, k * hdim:(k + 1) * hdim] = out


def _att1_proj2_body(adj_ref, rowfac_ref, colfac_ref, hpack_ref, hsum_ref,
                     w2_ref, ab2_ref, mask_ref, hpack2_ref, rowfac2_ref,
                     colfac2_ref, hsum2_ref, *, heads, hdim):
    i = pl.program_id(0)
    adjb = adj_ref[...].astype(jnp.bfloat16)
    mask_ref[...] = adjb.astype(jnp.float8_e5m2)
    n_src = adj_ref.shape[1]
    outs = []
    for k in range(heads):
        r = rowfac_ref[:, k:k + 1]
        q = colfac_ref[k:k + 1, :]
        w = jnp.maximum(q, r) * adjb
        acc = jnp.dot(w, hpack_ref[:, k * _STRIDE:k * _STRIDE + hdim + 1],
                      preferred_element_type=jnp.float32)
        num, z = acc[:, :hdim], acc[:, hdim:hdim + 1]
        hmean = hsum_ref[:, k * hdim:(k + 1) * hdim] * (1.0 / n_src)
        outs.append(jnp.maximum(jnp.where(z > 0, num / z, hmean), 0.0))
    o1 = jnp.concatenate(outs, axis=1)
    h2 = jnp.dot(o1, w2_ref[...], preferred_element_type=jnp.float32)
    g2 = jnp.dot(h2, ab2_ref[...], preferred_element_type=jnp.float32)
    gs2, gd2 = g2[:, :1], g2[:, 1:]
    rowfac2_ref[...] = jnp.exp((_ALPHA - 1.0) * gs2).astype(jnp.bfloat16)
    colfac2_ref[...] = jnp.exp((1.0 - _ALPHA) * gd2).astype(jnp.bfloat16).T
    fd2 = jnp.exp(_ALPHA * gd2)
    hdim2 = h2.shape[1]
    zeros = jnp.zeros((h2.shape[0], _STRIDE - hdim2 - 1), dtype=jnp.float32)
    hpack2_ref[...] = jnp.concatenate(
        [h2 * fd2, fd2, zeros], axis=1).astype(jnp.bfloat16)

    @pl.when(i == 0)
    def _init():
        hsum2_ref[...] = jnp.zeros_like(hsum2_ref)

    hsum2_ref[...] += jnp.sum(h2, axis=0, keepdims=True)


def _att1_proj2(adj, rowfac, colfac, hpack, hsum, w2c, ab2, heads, hdim,
                hdim2, tile):
    n = adj.shape[0]
    m2 = colfac.shape[0]
    mh = rowfac.shape[1]
    dp = hpack.shape[1]
    body = functools.partial(_att1_proj2_body, heads=heads, hdim=hdim)
    return pl.pallas_call(
        body,
        grid=(n // tile,),
        in_specs=[
            pl.BlockSpec((tile, n), lambda i: (i, 0)),
            pl.BlockSpec((tile, mh), lambda i: (i, 0)),
            pl.BlockSpec((m2, n), lambda i: (0, 0)),
            pl.BlockSpec((n, dp), lambda i: (0, 0)),
            pl.BlockSpec((1, heads * hdim), lambda i: (0, 0)),
            pl.BlockSpec((heads * hdim, hdim2), lambda i: (0, 0)),
            pl.BlockSpec((hdim2, 2), lambda i: (0, 0)),
        ],
        out_specs=[
            pl.BlockSpec((tile, n), lambda i: (i, 0)),
            pl.BlockSpec((tile, _STRIDE), lambda i: (i, 0)),
            pl.BlockSpec((tile, 1), lambda i: (i, 0)),
            pl.BlockSpec((1, tile), lambda i: (0, i)),
            pl.BlockSpec((1, hdim2), lambda i: (0, 0)),
        ],
        out_shape=[
            jax.ShapeDtypeStruct((n, n), jnp.float8_e5m2),
            jax.ShapeDtypeStruct((n, _STRIDE), jnp.bfloat16),
            jax.ShapeDtypeStruct((n, 1), jnp.bfloat16),
            jax.ShapeDtypeStruct((1, n), jnp.bfloat16),
            jax.ShapeDtypeStruct((1, hdim2), jnp.float32),
        ],
    )(adj, rowfac, colfac, hpack, hsum, w2c, ab2)


def _attention(adj, rowfac, colfac, hpack, hsum, heads, hdim, relu, tile,
               emit_mask=False):
    n = adj.shape[0]
    m2 = colfac.shape[0]
    mh = rowfac.shape[1]
    dp = hpack.shape[1]
    d = heads * hdim
    body = functools.partial(_att_body, heads=heads, hdim=hdim, relu=relu)
    out_specs = [pl.BlockSpec((tile, d), lambda i: (i, 0))]
    out_shape = [jax.ShapeDtypeStruct((n, d), jnp.float32)]
    if emit_mask:
        out_specs.append(pl.BlockSpec((tile, n), lambda i: (i, 0)))
        out_shape.append(jax.ShapeDtypeStruct((n, n), jnp.float8_e5m2))
    res = pl.pallas_call(
        body,
        grid=(n // tile,),
        in_specs=[
            pl.BlockSpec((tile, n), lambda i: (i, 0)),
            pl.BlockSpec((tile, mh), lambda i: (i, 0)),
            pl.BlockSpec((m2, n), lambda i: (0, 0)),
            pl.BlockSpec((n, dp), lambda i: (0, 0)),
            pl.BlockSpec((1, d), lambda i: (0, 0)),
        ],
        out_specs=out_specs,
        out_shape=out_shape,
        compiler_params=pltpu.CompilerParams(
            dimension_semantics=("parallel",)),
    )(adj, rowfac, colfac, hpack, hsum)
    return res if emit_mask else (res[0], None)


def kernel(x, adj, W1, a1_src, a1_dst, W2, a2_src, a2_dst):
    K, F_IN, H = W1.shape
    C = W2.shape[2]

    w1c = jnp.transpose(W1, (1, 0, 2)).reshape(F_IN, K * H)
    eye_k = jnp.eye(K, dtype=jnp.float32)
    A1 = jnp.einsum('ko,kj->koj', a1_src, eye_k).reshape(K * H, K)
    B1 = jnp.einsum('ko,kj->koj', a1_dst, eye_k).reshape(K * H, K)
    ab1 = jnp.concatenate([A1, B1], axis=1)
    w2c = W2.reshape(K * H, C)
    ab2 = jnp.concatenate([a2_src.T, a2_dst.T], axis=1)

    _, hp1, rf1, cf1, hs1 = _project(x, w1c, ab1, heads=K, hdim=H, tile=512)
    mask8, hp2, rf2, cf2, hs2 = _att1_proj2(
        adj, rf1, cf1, hp1, hs1, w2c, ab2, heads=K, hdim=H, hdim2=C,
        tile=1024)
    out, _ = _attention(mask8, rf2, cf2, hp2, hs2, heads=1, hdim=C,
                        relu=False, tile=1024)
    return out
